# baseline (speedup 1.0000x reference)
import functools

import jax
import jax.numpy as jnp
from jax import lax
from jax.experimental import pallas as pl
from jax.experimental.pallas import tpu as pltpu
from jax.experimental.pallas import tpu_sc as plsc

NC, NS = 2, 16
NW = NC * NS
CH = 128

F = 64
CW = 16


def _pad_rows(x, rows, value=0.0):
    return jnp.pad(x, ((0, rows - x.shape[0]), (0, 0)), constant_values=value)


def _make_gather(NP, EP, K, W):
    wchunk = EP // NW
    mesh = plsc.VectorSubcoreMesh(
        core_axis_name="c", subcore_axis_name="s", num_cores=NC, num_subcores=NS)

    @functools.partial(
        pl.kernel,
        out_type=jax.ShapeDtypeStruct((EP, W), jnp.float32),
        mesh=mesh,
        compiler_params=pltpu.CompilerParams(use_tc_tiling_on_sc=False),
        scratch_types=[
            pltpu.VMEM((K, CH), jnp.int32),
            pltpu.VMEM((wchunk, W), jnp.float32),
            pltpu.SemaphoreType.DMA,
        ],
    )
    def gather_kernel(table_hbm, idx_hbm, out_hbm, idx_v, rows_v, sem):
        wid = lax.axis_index("s") * NC + lax.axis_index("c")
        pltpu.sync_copy(idx_hbm.at[wid], idx_v)
        cps = [
            pltpu.async_copy(table_hbm.at[idx_v.at[j]],
                             rows_v.at[pl.ds(j * CH, CH)], sem)
            for j in range(K)
        ]
        for cp in cps:
            cp.wait()
        pltpu.sync_copy(rows_v, out_hbm.at[pl.ds(wid * wchunk, wchunk)])

    return gather_kernel


def _make_scatter(NP, EP, K, W):
    wchunk = EP // NW
    stripe = NP // NS
    mesh = plsc.VectorSubcoreMesh(
        core_axis_name="c", subcore_axis_name="s", num_cores=NC, num_subcores=NS)

    @functools.partial(
        pl.kernel,
        out_type=jax.ShapeDtypeStruct((NC, NP, W), jnp.float32),
        mesh=mesh,
        compiler_params=pltpu.CompilerParams(use_tc_tiling_on_sc=False),
        scratch_types=[
            pltpu.VMEM((K, CH), jnp.int32),
            pltpu.VMEM((2, CH, W), jnp.float32),
            pltpu.SemaphoreType.DMA,
            pltpu.SemaphoreType.DMA,
            pltpu.VMEM_SHARED((NP, W), jnp.float32),
        ],
    )
    def scatter_kernel(vals_hbm, idx_hbm, zeros_hbm, out_hbm,
                       idx_v, vals_v, sem0, sem1, acc):
        cid = lax.axis_index("c")
        sid = lax.axis_index("s")
        wid = sid * NC + cid
        pltpu.sync_copy(zeros_hbm.at[pl.ds(sid * stripe, stripe)],
                        acc.at[pl.ds(sid * stripe, stripe)])
        plsc.subcore_barrier()
        pltpu.sync_copy(idx_hbm.at[wid], idx_v)
        base = wid * wchunk
        sems = (sem0, sem1)
        loads = [
            pltpu.async_copy(vals_hbm.at[pl.ds(base + j * CH, CH)],
                             vals_v.at[j % 2], sems[j % 2])
            for j in range(min(2, K))
        ]
        for j in range(K):
            loads[j].wait()
            pltpu.sync_copy(vals_v.at[j % 2], acc.at[idx_v.at[j]], add=True)
            if j + 2 < K:
                loads.append(
                    pltpu.async_copy(vals_hbm.at[pl.ds(base + (j + 2) * CH, CH)],
                                     vals_v.at[j % 2], sems[j % 2]))
        plsc.subcore_barrier()
        pltpu.sync_copy(acc.at[pl.ds(sid * stripe, stripe)],
                        out_hbm.at[cid, pl.ds(sid * stripe, stripe)])

    return scatter_kernel


def _linrelu(x, W, b, BR):
    R, CIN = x.shape
    COUT = W.shape[1]

    def body(x_ref, w_ref, b_ref, o_ref):
        o_ref[...] = jax.nn.relu(
            jnp.dot(x_ref[...], w_ref[...],
                    preferred_element_type=jnp.float32,
                    precision=lax.Precision.HIGHEST) + b_ref[...])

    return pl.pallas_call(
        body,
        grid=(R // BR,),
        in_specs=[
            pl.BlockSpec((BR, CIN), lambda i: (i, 0)),
            pl.BlockSpec((CIN, COUT), lambda i: (0, 0)),
            pl.BlockSpec((1, COUT), lambda i: (0, 0)),
        ],
        out_specs=pl.BlockSpec((BR, COUT), lambda i: (i, 0)),
        out_shape=jax.ShapeDtypeStruct((R, COUT), jnp.float32),
    )(x, W, b.reshape(1, COUT))


def _msgs(xs, r, w2iko, B2, BR):
    EP = xs.shape[0]

    def body(x_ref, r_ref, w_ref, b2_ref, o_ref):
        x = x_ref[...]
        T = jnp.dot(x, w_ref[...], preferred_element_type=jnp.float32,
                    precision=lax.Precision.HIGHEST)
        rr = r_ref[...]
        acc = jnp.dot(x, b2_ref[...], preferred_element_type=jnp.float32,
                    precision=lax.Precision.HIGHEST)
        for k in range(F):
            acc = acc + rr[:, k:k + 1] * T[:, k * F:(k + 1) * F]
        o_ref[...] = acc

    return pl.pallas_call(
        body,
        grid=(EP // BR,),
        in_specs=[
            pl.BlockSpec((BR, F), lambda i: (i, 0)),
            pl.BlockSpec((BR, F), lambda i: (i, 0)),
            pl.BlockSpec((F, F * F), lambda i: (0, 0)),
            pl.BlockSpec((F, F), lambda i: (0, 0)),
        ],
        out_specs=pl.BlockSpec((BR, F), lambda i: (i, 0)),
        out_shape=jax.ShapeDtypeStruct((EP, F), jnp.float32),
    )(xs, r, w2iko, B2)


def _update(p, cnt, out, root, bias, BR):
    NP = out.shape[0]

    def body(p_ref, c_ref, x_ref, rt_ref, b_ref, o_ref):
        c = c_ref[0, :, 0:1] + c_ref[1, :, 0:1]
        inv = 1.0 / jnp.maximum(c, 1.0)
        aggr = (p_ref[0] + p_ref[1]) * inv
        o_ref[...] = jax.nn.relu(
            aggr + jnp.dot(x_ref[...], rt_ref[...],
                           preferred_element_type=jnp.float32,
                    precision=lax.Precision.HIGHEST) + b_ref[...])

    return pl.pallas_call(
        body,
        grid=(NP // BR,),
        in_specs=[
            pl.BlockSpec((NC, BR, F), lambda i: (0, i, 0)),
            pl.BlockSpec((NC, BR, CW), lambda i: (0, i, 0)),
            pl.BlockSpec((BR, F), lambda i: (i, 0)),
            pl.BlockSpec((F, F), lambda i: (0, 0)),
            pl.BlockSpec((1, F), lambda i: (0, 0)),
        ],
        out_specs=pl.BlockSpec((BR, F), lambda i: (i, 0)),
        out_shape=jax.ShapeDtypeStruct((NP, F), jnp.float32),
    )(p, cnt, out, root, bias.reshape(1, F))


def _masked_colsum(x, n_valid):
    NP = x.shape[0]

    def body(x_ref, o_ref):
        rid = lax.broadcasted_iota(jnp.int32, (NP, F), 0)
        o_ref[...] = jnp.sum(
            jnp.where(rid < n_valid, x_ref[...], 0.0), axis=0, keepdims=True)

    return pl.pallas_call(
        body,
        out_shape=jax.ShapeDtypeStruct((1, F), jnp.float32),
    )(x)


def _head(cat, bott_W, bott_b, lin1_W, lin1_b, lin2_W, lin2_b):
    def body(c_ref, bw_ref, bb_ref, w1_ref, b1_ref, w2_ref, b2_ref, o_ref):
        h = jax.nn.relu(
            jnp.dot(c_ref[...], bw_ref[...],
                    preferred_element_type=jnp.float32,
                    precision=lax.Precision.HIGHEST) + bb_ref[...])
        for _ in range(6):
            h = jax.nn.relu(
                jnp.dot(h, w1_ref[...],
                        preferred_element_type=jnp.float32,
                    precision=lax.Precision.HIGHEST) + b1_ref[...])
        o_ref[...] = jnp.dot(h, w2_ref[...],
                             preferred_element_type=jnp.float32,
                    precision=lax.Precision.HIGHEST) + b2_ref[...]

    return pl.pallas_call(
        body,
        out_shape=jax.ShapeDtypeStruct((1, 1), jnp.float32),
    )(cat, bott_W, bott_b.reshape(1, -1), lin1_W, lin1_b.reshape(1, -1),
      lin2_W, lin2_b.reshape(1, 1))


def _branch(x0, edge_index, edge_attr, lW, lb, w1, b1, w2, b2, root, bias,
            n, NP, EP, K, n_iters):
    src = edge_index[0]
    dst = edge_index[1]
    e = edge_index.shape[1]

    srcw = jnp.pad(src, (0, EP - e)).reshape(NW, K, CH)
    dstw = jnp.pad(dst, (0, EP - e), constant_values=n).reshape(NW, K, CH)

    x0p = _pad_rows(x0, NP)
    eap = _pad_rows(edge_attr, EP)

    w2iko = w2.reshape(F, F, F).transpose(1, 0, 2).reshape(F, F * F)
    B2 = b2.reshape(F, F)

    zeros_f = jnp.zeros((NP, F), jnp.float32)
    zeros_c = jnp.zeros((NP, CW), jnp.float32)
    ones_e = jnp.ones((EP, CW), jnp.float32)

    gather = _make_gather(NP, EP, K, F)
    scat_f = _make_scatter(NP, EP, K, F)
    scat_c = _make_scatter(NP, EP, K, CW)

    out = _linrelu(x0p, lW, lb, 512)
    r = _linrelu(eap, w1, b1, 512)
    cnt = scat_c(ones_e, dstw, zeros_c)

    for _ in range(n_iters):
        xs = gather(out, srcw)
        msgs = _msgs(xs, r, w2iko, B2, 512)
        p = scat_f(msgs, dstw, zeros_f)
        out = _update(p, cnt, out, root, bias, 512)

    return _masked_colsum(out, n)


def kernel(graph_x, graph_edge_index, graph_edge_attr, lg_x, lg_edge_index,
           lg_edge_attr, adduct, lin0_W, lin0_b, nn1_W, nn1_b, nn2_W, nn2_b,
           conv_root, conv_bias, lin0lg_W, lin0lg_b, nnlg1_W, nnlg1_b,
           nnlg2_W, nnlg2_b, convlg_root, convlg_bias, bott_W, bott_b,
           lin1_W, lin1_b, lin2_W, lin2_b):
    sum_g = _branch(graph_x, graph_edge_index, graph_edge_attr,
                    lin0_W, lin0_b, nn1_W, nn1_b, nn2_W, nn2_b,
                    conv_root, conv_bias,
                    n=10000, NP=10240, EP=20480, K=5, n_iters=3)
    sum_lg = _branch(lg_x, lg_edge_index, lg_edge_attr,
                     lin0lg_W, lin0lg_b, nnlg1_W, nnlg1_b, nnlg2_W, nnlg2_b,
                     convlg_root, convlg_bias,
                     n=20000, NP=20480, EP=32768, K=8, n_iters=3)

    cat = jnp.concatenate([sum_g[0], sum_lg[0], adduct])[None, :]
    res = _head(cat, bott_W, bott_b, lin1_W, lin1_b, lin2_W, lin2_b)
    return res[0]


# speedup vs baseline: 1.0421x; 1.0421x over previous
import functools

import jax
import jax.numpy as jnp
from jax import lax
from jax.experimental import pallas as pl
from jax.experimental.pallas import tpu as pltpu
from jax.experimental.pallas import tpu_sc as plsc

NC, NS = 2, 16
NW = NC * NS
CH = 128

F = 64
CW = 16


def _pad_rows(x, rows, value=0.0):
    return jnp.pad(x, ((0, rows - x.shape[0]), (0, 0)), constant_values=value)


def _make_gather(NP, EP, K, W):
    wchunk = EP // NW
    mesh = plsc.VectorSubcoreMesh(
        core_axis_name="c", subcore_axis_name="s", num_cores=NC, num_subcores=NS)

    @functools.partial(
        pl.kernel,
        out_type=jax.ShapeDtypeStruct((EP, W), jnp.float32),
        mesh=mesh,
        compiler_params=pltpu.CompilerParams(use_tc_tiling_on_sc=False),
        scratch_types=[
            pltpu.VMEM((K, CH), jnp.int32),
            pltpu.VMEM((wchunk, W), jnp.float32),
            pltpu.SemaphoreType.DMA,
        ],
    )
    def gather_kernel(table_hbm, idx_hbm, out_hbm, idx_v, rows_v, sem):
        wid = lax.axis_index("s") * NC + lax.axis_index("c")
        pltpu.sync_copy(idx_hbm.at[wid], idx_v)
        cps = [
            pltpu.async_copy(table_hbm.at[idx_v.at[j]],
                             rows_v.at[pl.ds(j * CH, CH)], sem)
            for j in range(K)
        ]
        for cp in cps:
            cp.wait()
        pltpu.sync_copy(rows_v, out_hbm.at[pl.ds(wid * wchunk, wchunk)])

    return gather_kernel


def _make_scatter(NP, EP, K, W):
    wchunk = EP // NW
    stripe = NP // NS
    mesh = plsc.VectorSubcoreMesh(
        core_axis_name="c", subcore_axis_name="s", num_cores=NC, num_subcores=NS)

    @functools.partial(
        pl.kernel,
        out_type=jax.ShapeDtypeStruct((NC, NP, W), jnp.float32),
        mesh=mesh,
        compiler_params=pltpu.CompilerParams(use_tc_tiling_on_sc=False),
        scratch_types=[
            pltpu.VMEM((K, CH), jnp.int32),
            pltpu.VMEM((2, CH, W), jnp.float32),
            pltpu.SemaphoreType.DMA,
            pltpu.SemaphoreType.DMA,
            pltpu.VMEM_SHARED((NP, W), jnp.float32),
        ],
    )
    def scatter_kernel(vals_hbm, idx_hbm, zeros_hbm, out_hbm,
                       idx_v, vals_v, sem0, sem1, acc):
        cid = lax.axis_index("c")
        sid = lax.axis_index("s")
        wid = sid * NC + cid
        pltpu.sync_copy(zeros_hbm.at[pl.ds(sid * stripe, stripe)],
                        acc.at[pl.ds(sid * stripe, stripe)])
        plsc.subcore_barrier()
        pltpu.sync_copy(idx_hbm.at[wid], idx_v)
        base = wid * wchunk
        sems = (sem0, sem1)
        loads = [
            pltpu.async_copy(vals_hbm.at[pl.ds(base + j * CH, CH)],
                             vals_v.at[j % 2], sems[j % 2])
            for j in range(min(2, K))
        ]
        for j in range(K):
            loads[j].wait()
            pltpu.sync_copy(vals_v.at[j % 2], acc.at[idx_v.at[j]], add=True)
            if j + 2 < K:
                loads.append(
                    pltpu.async_copy(vals_hbm.at[pl.ds(base + (j + 2) * CH, CH)],
                                     vals_v.at[j % 2], sems[j % 2]))
        plsc.subcore_barrier()
        pltpu.sync_copy(acc.at[pl.ds(sid * stripe, stripe)],
                        out_hbm.at[cid, pl.ds(sid * stripe, stripe)])

    return scatter_kernel


def _linrelu(x, W, b, BR):
    R, CIN = x.shape
    COUT = W.shape[1]

    def body(x_ref, w_ref, b_ref, o_ref):
        o_ref[...] = jax.nn.relu(
            jnp.dot(x_ref[...], w_ref[...],
                    preferred_element_type=jnp.float32,
                    precision=lax.Precision.HIGHEST) + b_ref[...])

    return pl.pallas_call(
        body,
        grid=(R // BR,),
        in_specs=[
            pl.BlockSpec((BR, CIN), lambda i: (i, 0)),
            pl.BlockSpec((CIN, COUT), lambda i: (0, 0)),
            pl.BlockSpec((1, COUT), lambda i: (0, 0)),
        ],
        out_specs=pl.BlockSpec((BR, COUT), lambda i: (i, 0)),
        out_shape=jax.ShapeDtypeStruct((R, COUT), jnp.float32),
    )(x, W, b.reshape(1, COUT))


def _msgs(xs, r, w2iko, B2, BR):
    EP = xs.shape[0]

    def body(x_ref, r_ref, w_ref, b2_ref, o_ref):
        x = x_ref[...]
        w = w_ref[...]
        xh = x.astype(jnp.bfloat16)
        xl = (x - xh.astype(jnp.float32)).astype(jnp.bfloat16)
        wh = w.astype(jnp.bfloat16)
        wl = (w - wh.astype(jnp.float32)).astype(jnp.bfloat16)
        T = (jnp.dot(xh, wh, preferred_element_type=jnp.float32)
             + jnp.dot(xh, wl, preferred_element_type=jnp.float32)
             + jnp.dot(xl, wh, preferred_element_type=jnp.float32))
        rr = r_ref[...]
        acc = jnp.dot(x, b2_ref[...], preferred_element_type=jnp.float32,
                    precision=lax.Precision.HIGHEST)
        for k in range(F):
            acc = acc + rr[:, k:k + 1] * T[:, k * F:(k + 1) * F]
        o_ref[...] = acc

    return pl.pallas_call(
        body,
        grid=(EP // BR,),
        in_specs=[
            pl.BlockSpec((BR, F), lambda i: (i, 0)),
            pl.BlockSpec((BR, F), lambda i: (i, 0)),
            pl.BlockSpec((F, F * F), lambda i: (0, 0)),
            pl.BlockSpec((F, F), lambda i: (0, 0)),
        ],
        out_specs=pl.BlockSpec((BR, F), lambda i: (i, 0)),
        out_shape=jax.ShapeDtypeStruct((EP, F), jnp.float32),
    )(xs, r, w2iko, B2)


def _update(p, cnt, out, root, bias, BR):
    NP = out.shape[0]

    def body(p_ref, c_ref, x_ref, rt_ref, b_ref, o_ref):
        c = c_ref[0, :, 0:1] + c_ref[1, :, 0:1]
        inv = 1.0 / jnp.maximum(c, 1.0)
        aggr = (p_ref[0] + p_ref[1]) * inv
        o_ref[...] = jax.nn.relu(
            aggr + jnp.dot(x_ref[...], rt_ref[...],
                           preferred_element_type=jnp.float32,
                    precision=lax.Precision.HIGHEST) + b_ref[...])

    return pl.pallas_call(
        body,
        grid=(NP // BR,),
        in_specs=[
            pl.BlockSpec((NC, BR, F), lambda i: (0, i, 0)),
            pl.BlockSpec((NC, BR, CW), lambda i: (0, i, 0)),
            pl.BlockSpec((BR, F), lambda i: (i, 0)),
            pl.BlockSpec((F, F), lambda i: (0, 0)),
            pl.BlockSpec((1, F), lambda i: (0, 0)),
        ],
        out_specs=pl.BlockSpec((BR, F), lambda i: (i, 0)),
        out_shape=jax.ShapeDtypeStruct((NP, F), jnp.float32),
    )(p, cnt, out, root, bias.reshape(1, F))


def _masked_colsum(x, n_valid):
    NP = x.shape[0]

    def body(x_ref, o_ref):
        rid = lax.broadcasted_iota(jnp.int32, (NP, F), 0)
        o_ref[...] = jnp.sum(
            jnp.where(rid < n_valid, x_ref[...], 0.0), axis=0, keepdims=True)

    return pl.pallas_call(
        body,
        out_shape=jax.ShapeDtypeStruct((1, F), jnp.float32),
    )(x)


def _head(cat, bott_W, bott_b, lin1_W, lin1_b, lin2_W, lin2_b):
    def body(c_ref, bw_ref, bb_ref, w1_ref, b1_ref, w2_ref, b2_ref, o_ref):
        h = jax.nn.relu(
            jnp.dot(c_ref[...], bw_ref[...],
                    preferred_element_type=jnp.float32,
                    precision=lax.Precision.HIGHEST) + bb_ref[...])
        for _ in range(6):
            h = jax.nn.relu(
                jnp.dot(h, w1_ref[...],
                        preferred_element_type=jnp.float32,
                    precision=lax.Precision.HIGHEST) + b1_ref[...])
        o_ref[...] = jnp.dot(h, w2_ref[...],
                             preferred_element_type=jnp.float32,
                    precision=lax.Precision.HIGHEST) + b2_ref[...]

    return pl.pallas_call(
        body,
        out_shape=jax.ShapeDtypeStruct((1, 1), jnp.float32),
    )(cat, bott_W, bott_b.reshape(1, -1), lin1_W, lin1_b.reshape(1, -1),
      lin2_W, lin2_b.reshape(1, 1))


def _branch(x0, edge_index, edge_attr, lW, lb, w1, b1, w2, b2, root, bias,
            n, NP, EP, K, n_iters):
    src = edge_index[0]
    dst = edge_index[1]
    e = edge_index.shape[1]

    srcw = jnp.pad(src, (0, EP - e)).reshape(NW, K, CH)
    dstw = jnp.pad(dst, (0, EP - e), constant_values=n).reshape(NW, K, CH)

    x0p = _pad_rows(x0, NP)
    eap = _pad_rows(edge_attr, EP)

    w2iko = w2.reshape(F, F, F).transpose(1, 0, 2).reshape(F, F * F)
    B2 = b2.reshape(F, F)

    zeros_f = jnp.zeros((NP, F), jnp.float32)
    zeros_c = jnp.zeros((NP, CW), jnp.float32)
    ones_e = jnp.ones((EP, CW), jnp.float32)

    gather = _make_gather(NP, EP, K, F)
    scat_f = _make_scatter(NP, EP, K, F)
    scat_c = _make_scatter(NP, EP, K, CW)

    out = _linrelu(x0p, lW, lb, 512)
    r = _linrelu(eap, w1, b1, 512)
    cnt = scat_c(ones_e, dstw, zeros_c)

    for _ in range(n_iters):
        xs = gather(out, srcw)
        msgs = _msgs(xs, r, w2iko, B2, 512)
        p = scat_f(msgs, dstw, zeros_f)
        out = _update(p, cnt, out, root, bias, 512)

    return _masked_colsum(out, n)


def kernel(graph_x, graph_edge_index, graph_edge_attr, lg_x, lg_edge_index,
           lg_edge_attr, adduct, lin0_W, lin0_b, nn1_W, nn1_b, nn2_W, nn2_b,
           conv_root, conv_bias, lin0lg_W, lin0lg_b, nnlg1_W, nnlg1_b,
           nnlg2_W, nnlg2_b, convlg_root, convlg_bias, bott_W, bott_b,
           lin1_W, lin1_b, lin2_W, lin2_b):
    sum_g = _branch(graph_x, graph_edge_index, graph_edge_attr,
                    lin0_W, lin0_b, nn1_W, nn1_b, nn2_W, nn2_b,
                    conv_root, conv_bias,
                    n=10000, NP=10240, EP=20480, K=5, n_iters=3)
    sum_lg = _branch(lg_x, lg_edge_index, lg_edge_attr,
                     lin0lg_W, lin0lg_b, nnlg1_W, nnlg1_b, nnlg2_W, nnlg2_b,
                     convlg_root, convlg_bias,
                     n=20000, NP=20480, EP=32768, K=8, n_iters=3)

    cat = jnp.concatenate([sum_g[0], sum_lg[0], adduct])[None, :]
    res = _head(cat, bott_W, bott_b, lin1_W, lin1_b, lin2_W, lin2_b)
    return res[0]
